# baseline (speedup 1.0000x reference)
import functools

import jax
import jax.numpy as jnp
from jax import lax
from jax.experimental import pallas as pl
from jax.experimental.pallas import tpu as pltpu
from jax.experimental.pallas import tpu_sc as plsc

DIM = 64
COND_DIM = 256
TILE_ROWS = 8

_CHUNK = 32
_LANES = 16


def _sc_workers():
    try:
        info = plsc.get_sparse_core_info()
        return info.num_cores, info.num_subcores
    except Exception:
        return 2, 16


def _make_sc_gather(batch, dim):
    nc, ns = _sc_workers()
    nw = nc * ns
    assert batch % (8 * nw) == 0
    b_per_w = batch // nw
    assert b_per_w % _CHUNK == 0
    n_chunks = b_per_w // _CHUNK
    mesh = plsc.VectorSubcoreMesh(core_axis_name="c", subcore_axis_name="s")

    @functools.partial(
        pl.kernel,
        mesh=mesh,
        out_type=jax.ShapeDtypeStruct((batch, dim), jnp.float32),
        scratch_types=[
            pltpu.VMEM((b_per_w,), jnp.int32),
            pltpu.VMEM((2, _CHUNK, TILE_ROWS, dim), jnp.float32),
            pltpu.VMEM((2, _CHUNK, dim), jnp.float32),
            pltpu.SemaphoreType.DMA,
            pltpu.SemaphoreType.DMA,
        ],
    )
    def gather(table_hbm, idx_hbm, out_hbm, idx_v, tiles_v, rows_v, sem_a, sem_b):
        sems = (sem_a, sem_b)
        wid = lax.axis_index("s") * nc + lax.axis_index("c")
        base = wid * b_per_w
        pltpu.sync_copy(idx_hbm.at[pl.ds(base, b_per_w)], idx_v)

        def fire(off, buf):
            off = pl.multiple_of(off, _CHUNK)
            for g in range(_CHUNK // _LANES):
                v = idx_v[pl.ds(off + g * _LANES, _LANES)]
                tv = lax.shift_right_logical(v, 3)
                for l in range(_LANES):
                    pltpu.async_copy(
                        table_hbm.at[tv[l]],
                        tiles_v.at[buf, g * _LANES + l],
                        sems[buf],
                    )

        def process(off, buf):
            off = pl.multiple_of(off, _CHUNK)
            for k in range(_CHUNK):
                pltpu.make_async_copy(
                    table_hbm.at[0], tiles_v.at[buf, k], sems[buf]
                ).wait()
            for g in range(_CHUNK // _LANES):
                v = idx_v[pl.ds(off + g * _LANES, _LANES)]
                sv = v & 7
                for l in range(_LANES):
                    k = g * _LANES + l
                    for q in range(dim // _LANES):
                        sl = pl.ds(q * _LANES, _LANES)
                        rows_v[buf, k, sl] = tiles_v[buf, k, sv[l], sl]
            pltpu.sync_copy(
                rows_v.at[buf], out_hbm.at[pl.ds(base + off, _CHUNK)]
            )

        fire(0, 0)

        def body(p, carry):
            o = p * (2 * _CHUNK)
            fire(o + _CHUNK, 1)
            process(o, 0)
            fire(o + 2 * _CHUNK, 0)
            process(o + _CHUNK, 1)
            return carry

        lax.fori_loop(0, n_chunks // 2 - 1, body, 0, unroll=False)
        o_last = (n_chunks - 2) * _CHUNK
        fire(o_last + _CHUNK, 1)
        process(o_last, 0)
        process(o_last + _CHUNK, 1)

    return gather


def _relayout_body(xt_ref, o_ref):
    x = xt_ref[...]
    eye = jnp.eye(DIM, dtype=jnp.float32)
    o_ref[...] = jax.lax.dot_general(
        x, eye, (((0,), (0,)), ((), ())), preferred_element_type=jnp.float32
    )


def _make_tc_relayout(rows, blk):
    nblk = (rows + blk - 1) // blk
    return pl.pallas_call(
        _relayout_body,
        grid=(nblk,),
        in_specs=[pl.BlockSpec((DIM, blk), lambda i: (0, i))],
        out_specs=pl.BlockSpec((blk, DIM), lambda i: (i, 0)),
        out_shape=jax.ShapeDtypeStruct((rows, DIM), jnp.float32),
    )


def _mlp_body(x_ref, w_ref, b_ref, o_ref):
    y = jnp.dot(x_ref[...], w_ref[...], preferred_element_type=jnp.float32)
    y = y + b_ref[...]
    o_ref[...] = y * jax.nn.sigmoid(y)


def _make_tc_mlp(batch, dim, cond_dim, blk):
    assert batch % blk == 0
    return pl.pallas_call(
        _mlp_body,
        grid=(batch // blk,),
        in_specs=[
            pl.BlockSpec((blk, dim), lambda i: (i, 0)),
            pl.BlockSpec((dim, cond_dim), lambda i: (0, 0)),
            pl.BlockSpec((1, cond_dim), lambda i: (0, 0)),
        ],
        out_specs=pl.BlockSpec((blk, cond_dim), lambda i: (i, 0)),
        out_shape=jax.ShapeDtypeStruct((batch, cond_dim), jnp.float32),
    )


def kernel(condition, cond_embed, W, b):
    idx = condition.reshape(-1).astype(jnp.int32)
    batch = idx.shape[0]
    rows = cond_embed.shape[0]
    table_rm = _make_tc_relayout(rows, blk=8192)(cond_embed.T)
    table3 = table_rm.reshape(-1, TILE_ROWS, DIM)
    gathered = _make_sc_gather(batch, DIM)(table3, idx)
    mlp = _make_tc_mlp(batch, DIM, COND_DIM, blk=4096)
    return mlp(gathered, W, b.reshape(1, COND_DIM))


# speedup vs baseline: 1.1851x; 1.1851x over previous
import functools

import jax
import jax.numpy as jnp
from jax import lax
from jax.experimental import pallas as pl
from jax.experimental.pallas import tpu as pltpu
from jax.experimental.pallas import tpu_sc as plsc

DIM = 64
COND_DIM = 256
TILE_ROWS = 8

_CHUNK = 32
_LANES = 16


def _sc_workers():
    try:
        info = plsc.get_sparse_core_info()
        return info.num_cores, info.num_subcores
    except Exception:
        return 2, 16


def _make_sc_gather(batch, dim):
    nc, ns = _sc_workers()
    nw = nc * ns
    assert batch % (8 * nw) == 0
    b_per_w = batch // nw
    assert b_per_w % _CHUNK == 0
    n_chunks = b_per_w // _CHUNK
    mesh = plsc.VectorSubcoreMesh(core_axis_name="c", subcore_axis_name="s")

    @functools.partial(
        pl.kernel,
        mesh=mesh,
        out_type=jax.ShapeDtypeStruct((batch, dim), jnp.float32),
        scratch_types=[
            pltpu.VMEM((b_per_w,), jnp.int32),
            pltpu.VMEM((2, _CHUNK, TILE_ROWS, dim), jnp.float32),
            pltpu.VMEM((2, _CHUNK, dim), jnp.float32),
            pltpu.SemaphoreType.DMA,
            pltpu.SemaphoreType.DMA,
        ],
    )
    def gather(table_hbm, idx_hbm, out_hbm, idx_v, tiles_v, rows_v, sem_a, sem_b):
        sems = (sem_a, sem_b)
        wid = lax.axis_index("s") * nc + lax.axis_index("c")
        base = wid * b_per_w
        pltpu.sync_copy(idx_hbm.at[pl.ds(base, b_per_w)], idx_v)

        def fire(off, buf):
            off = pl.multiple_of(off, _CHUNK)
            for g in range(_CHUNK // _LANES):
                v = idx_v[pl.ds(off + g * _LANES, _LANES)]
                tv = lax.shift_right_logical(v, 3)
                for l in range(_LANES):
                    pltpu.async_copy(
                        table_hbm.at[tv[l]],
                        tiles_v.at[buf, g * _LANES + l],
                        sems[buf],
                    )

        def process(off, buf):
            off = pl.multiple_of(off, _CHUNK)
            for k in range(_CHUNK):
                pltpu.make_async_copy(
                    table_hbm.at[0], tiles_v.at[buf, k], sems[buf]
                ).wait()
            for g in range(_CHUNK // _LANES):
                v = idx_v[pl.ds(off + g * _LANES, _LANES)]
                sv = v & 7
                for l in range(_LANES):
                    k = g * _LANES + l
                    for q in range(dim // _LANES):
                        sl = pl.ds(q * _LANES, _LANES)
                        rows_v[buf, k, sl] = tiles_v[buf, k, sv[l], sl]
            pltpu.sync_copy(
                rows_v.at[buf], out_hbm.at[pl.ds(base + off, _CHUNK)]
            )

        fire(0, 0)

        def body(p, carry):
            o = p * (2 * _CHUNK)
            fire(o + _CHUNK, 1)
            process(o, 0)
            fire(o + 2 * _CHUNK, 0)
            process(o + _CHUNK, 1)
            return carry

        lax.fori_loop(0, n_chunks // 2 - 1, body, 0, unroll=False)
        o_last = (n_chunks - 2) * _CHUNK
        fire(o_last + _CHUNK, 1)
        process(o_last, 0)
        process(o_last + _CHUNK, 1)

    return gather


def _mlp_body(x_ref, w_ref, b_ref, o_ref):
    y = jnp.dot(x_ref[...], w_ref[...], preferred_element_type=jnp.float32)
    y = y + b_ref[...]
    o_ref[...] = y * jax.nn.sigmoid(y)


def _make_tc_mlp(batch, dim, cond_dim, blk):
    assert batch % blk == 0
    return pl.pallas_call(
        _mlp_body,
        grid=(batch // blk,),
        in_specs=[
            pl.BlockSpec((blk, dim), lambda i: (i, 0)),
            pl.BlockSpec((dim, cond_dim), lambda i: (0, 0)),
            pl.BlockSpec((1, cond_dim), lambda i: (0, 0)),
        ],
        out_specs=pl.BlockSpec((blk, cond_dim), lambda i: (i, 0)),
        out_shape=jax.ShapeDtypeStruct((batch, cond_dim), jnp.float32),
    )


def kernel(condition, cond_embed, W, b):
    idx = condition.reshape(-1).astype(jnp.int32)
    batch = idx.shape[0]
    table3 = cond_embed.reshape(-1, TILE_ROWS, DIM)
    gathered = _make_sc_gather(batch, DIM)(table3, idx)
    mlp = _make_tc_mlp(batch, DIM, COND_DIM, blk=4096)
    return mlp(gathered, W, b.reshape(1, COND_DIM))
